# baseline (speedup 1.0000x reference)
import functools

import jax
import jax.numpy as jnp
from jax import lax
from jax.experimental import pallas as pl
from jax.experimental.pallas import tpu as pltpu
from jax.experimental.pallas import tpu_sc as plsc

N = 50000
E = 800000
D_IN = 128
D_H = 64
D_OUT = 128

NC = 2
NS = 16
NW = NC * NS
LANES = 128
CHUNKS = 6272
EPAD = CHUNKS * LANES
PT_CHUNKS = CHUNKS // NW
STAGE = 49

HALF = N // 2
NPAD_H = 25600
ROWS_H = NPAD_H // NS
TRASH = HALF

AGG_K = 3
SEG_CHUNKS = 198
SEGCAP = SEG_CHUNKS * LANES

DEG_NPAD = 51200
DEG_ROWS = DEG_NPAD // NS
DEG_K = 4
DEG_GROUPS = PT_CHUNKS // DEG_K

_mesh = plsc.VectorSubcoreMesh(
    core_axis_name="c", subcore_axis_name="s", num_cores=NC, num_subcores=NS)
_lin = pltpu.CompilerParams(use_tc_tiling_on_sc=False)
_lin_nl = pltpu.CompilerParams(
    use_tc_tiling_on_sc=False, needs_layout_passes=False)


@functools.partial(
    pl.kernel,
    out_type=(jax.ShapeDtypeStruct((NC, NW, SEGCAP), jnp.int32),
              jax.ShapeDtypeStruct((NC, NW, SEGCAP), jnp.int32),
              jax.ShapeDtypeStruct((NC, NW, 16), jnp.int32)),
    mesh=_mesh,
    scratch_types=[
        pltpu.VMEM((STAGE * LANES,), jnp.int32),
        pltpu.VMEM((STAGE * LANES,), jnp.int32),
        pltpu.VMEM((SEGCAP + 16,), jnp.int32),
        pltpu.VMEM((SEGCAP + 16,), jnp.int32),
        pltpu.VMEM((SEGCAP + 16,), jnp.int32),
        pltpu.VMEM((SEGCAP + 16,), jnp.int32),
        pltpu.VMEM((16,), jnp.int32),
    ],
    compiler_params=_lin_nl,
)
def _sc_partition(src_hbm, dst_hbm, osrc_hbm, odst_hbm, ocnt_hbm,
                  ssrc, sdst, cs0, cd0, cs1, cd1, cbuf):
  c = lax.axis_index("c")
  s = lax.axis_index("s")
  t = c * NS + s
  base = t * PT_CHUNKS

  kround = AGG_K * LANES
  trash16 = jnp.full((16,), SEGCAP, jnp.int32)

  def stage_group(g, offs):
    gb = base * LANES + g * (STAGE * LANES)
    pltpu.sync_copy(src_hbm.at[pl.ds(gb, STAGE * LANES)], ssrc)
    pltpu.sync_copy(dst_hbm.at[pl.ds(gb, STAGE * LANES)], sdst)

    def vec(i, offs):
      o0, o1 = offs
      sv = ssrc[pl.ds(i * 16, 16)]
      dv = sdst[pl.ds(i * 16, 16)]
      m0 = dv < HALF
      m0i = m0.astype(jnp.int32)
      pos0 = jnp.cumsum(m0i)
      pos1 = jnp.cumsum(1 - m0i)
      idx0 = jnp.where(m0, o0 + pos0 - 1, trash16)
      idx1 = jnp.where(m0, trash16, o1 + pos1 - 1)
      plsc.store_scatter(cs0, [idx0], sv)
      plsc.store_scatter(cd0, [idx0], dv)
      plsc.store_scatter(cs1, [idx1], sv)
      plsc.store_scatter(cd1, [idx1], dv - HALF)
      pc = plsc.all_reduce_population_count(m0)
      return (o0 + pc, o1 + (16 - pc))

    return lax.fori_loop(0, STAGE * 8, vec, offs)

  z16i = jnp.zeros((16,), jnp.int32)
  o0v, o1v = lax.fori_loop(0, PT_CHUNKS // STAGE, stage_group, (z16i, z16i))

  zsrc = jnp.zeros((16,), jnp.int32)
  tdst = jnp.full((16,), TRASH, jnp.int32)

  def pad_half(csrc, cdst, offv):
    off = offv[0]
    csrc[pl.ds(off, 16)] = zsrc
    cdst[pl.ds(off, 16)] = tdst
    o16 = ((off + 15) // 16) * 16
    end = ((off + kround - 1) // kround) * kround

    def padv(k, carry):
      csrc[pl.ds(o16 + k * 16, 16)] = zsrc
      cdst[pl.ds(o16 + k * 16, 16)] = tdst
      return carry

    lax.fori_loop(0, (end - o16) // 16, padv, 0)
    return jax.lax.shift_right_logical(offv + (LANES - 1), 7)

  ng0v = pad_half(cs0, cd0, o0v)
  ng1v = pad_half(cs1, cd1, o1v)

  cbuf[...] = ng0v
  pltpu.sync_copy(cbuf, ocnt_hbm.at[0, t])
  cbuf[...] = ng1v
  pltpu.sync_copy(cbuf, ocnt_hbm.at[1, t])
  pltpu.sync_copy(cs0.at[pl.ds(0, SEGCAP)], osrc_hbm.at[0, t])
  pltpu.sync_copy(cd0.at[pl.ds(0, SEGCAP)], odst_hbm.at[0, t])
  pltpu.sync_copy(cs1.at[pl.ds(0, SEGCAP)], osrc_hbm.at[1, t])
  pltpu.sync_copy(cd1.at[pl.ds(0, SEGCAP)], odst_hbm.at[1, t])


@functools.partial(
    pl.kernel,
    out_type=(jax.ShapeDtypeStruct((DEG_NPAD,), jnp.float32),
              jax.ShapeDtypeStruct((DEG_NPAD,), jnp.float32)),
    mesh=_mesh,
    scratch_types=[
        pltpu.VMEM((DEG_K, LANES), jnp.int32),
        pltpu.VMEM((LANES,), jnp.float32),
        pltpu.VMEM((DEG_ROWS,), jnp.float32),
        pltpu.VMEM_SHARED((DEG_NPAD,), jnp.float32),
        pltpu.SemaphoreType.DMA,
    ],
    compiler_params=_lin,
)
def _sc_degree(dst_hbm, out0_hbm, out1_hbm, didx, ones, zbuf, acc, ssem):
  c = lax.axis_index("c")
  s = lax.axis_index("s")
  one16 = jnp.ones((16,), jnp.float32)
  z16 = jnp.zeros((16,), jnp.float32)

  def zb_body(i, carry):
    zbuf[pl.ds(i * 16, 16)] = z16
    return carry

  lax.fori_loop(0, DEG_ROWS // 16, zb_body, 0)
  for i in range(LANES // 16):
    ones[pl.ds(i * 16, 16)] = one16

  pltpu.sync_copy(zbuf, acc.at[pl.ds(s * DEG_ROWS, DEG_ROWS)])
  plsc.subcore_barrier()

  base = (c * NS + s) * PT_CHUNKS

  def group(g, carry):
    gb = base + g * DEG_K
    pltpu.sync_copy(dst_hbm.at[pl.ds(gb, DEG_K)], didx)
    handles = []
    for b in range(DEG_K):
      handles.append(
          pltpu.async_copy(ones, acc.at[didx.at[b]], ssem, add=True))
    for h in handles:
      h.wait()
    return carry

  lax.fori_loop(0, DEG_GROUPS, group, 0)
  plsc.subcore_barrier()
  sl = pl.ds(s * DEG_ROWS, DEG_ROWS)

  @pl.when(c == 0)
  def _():
    pltpu.sync_copy(acc.at[sl], out0_hbm.at[sl])

  @pl.when(c == 1)
  def _():
    pltpu.sync_copy(acc.at[sl], out1_hbm.at[sl])


@functools.partial(
    pl.kernel,
    out_type=jax.ShapeDtypeStruct((NC, NPAD_H, D_H), jnp.float32),
    mesh=_mesh,
    scratch_types=[
        pltpu.VMEM((AGG_K, LANES), jnp.int32),
        pltpu.VMEM((AGG_K, LANES), jnp.int32),
        pltpu.VMEM((AGG_K, LANES, D_H), jnp.float32),
        pltpu.VMEM((16,), jnp.int32),
        pltpu.VMEM_SHARED((NPAD_H, D_H), jnp.float32),
        pltpu.SemaphoreType.DMA,
        pltpu.SemaphoreType.DMA,
    ],
    compiler_params=_lin,
)
def _sc_agg(table_hbm, src_hbm, dst_hbm, cnt_hbm, out_hbm,
            sidx, didx, rows, cbuf, acc, gsem, ssem):
  c = lax.axis_index("c")
  s = lax.axis_index("s")

  z16 = jnp.zeros((16,), jnp.float32)

  def zrow(i, carry):
    for q in range(D_H // 16):
      rows[0, i, pl.ds(q * 16, 16)] = z16
    return carry

  lax.fori_loop(0, 64, zrow, 0)

  def zcopy(k, carry):
    pltpu.sync_copy(rows.at[0, pl.ds(0, 64)],
                    acc.at[pl.ds(s * ROWS_H + k * 64, 64)])
    return carry

  lax.fori_loop(0, ROWS_H // 64, zcopy, 0)
  plsc.subcore_barrier()

  for j in range(2):
    seg = 2 * s + j
    pltpu.sync_copy(cnt_hbm.at[c, seg], cbuf)
    ngc = cbuf[...][0]
    ng = (ngc + AGG_K - 1) // AGG_K

    def group(g, carry):
      gb = g * AGG_K
      pltpu.sync_copy(src_hbm.at[c, seg, pl.ds(gb, AGG_K)], sidx)
      pltpu.sync_copy(dst_hbm.at[c, seg, pl.ds(gb, AGG_K)], didx)
      gh = []
      for b in range(AGG_K):
        gh.append(pltpu.async_copy(table_hbm.at[sidx.at[b]], rows.at[b], gsem))
      for h in gh:
        h.wait()
      sh = []
      for b in range(AGG_K):
        sh.append(pltpu.async_copy(rows.at[b], acc.at[didx.at[b]], ssem,
                                   add=True))
      for h in sh:
        h.wait()
      return carry

    lax.fori_loop(0, ng, group, 0)

  plsc.subcore_barrier()
  pltpu.sync_copy(acc.at[pl.ds(s * ROWS_H, ROWS_H)],
                  out_hbm.at[c, pl.ds(s * ROWS_H, ROWS_H)])


_R = 2000
_R2 = 1000
_G2 = N // _R2
_HB = _G2 // 2


def _b1_body(x_ref, w_ref, d0_ref, d1_ref, g_ref, dinv_ref):
  deg = d0_ref[...] + d1_ref[...] + 1.0
  dinv = lax.rsqrt(deg)
  h = jnp.dot(x_ref[...], w_ref[...], preferred_element_type=jnp.float32)
  g_ref[...] = h * dinv
  dinv_ref[...] = dinv


def _tc_b1(x, w_in, d0, d1):
  return pl.pallas_call(
      _b1_body,
      grid=(N // _R,),
      in_specs=[
          pl.BlockSpec((_R, D_IN), lambda i: (i, 0)),
          pl.BlockSpec((D_IN, D_H), lambda i: (0, 0)),
          pl.BlockSpec((_R, 1), lambda i: (i, 0)),
          pl.BlockSpec((_R, 1), lambda i: (i, 0)),
      ],
      out_specs=[
          pl.BlockSpec((_R, D_H), lambda i: (i, 0)),
          pl.BlockSpec((_R, 1), lambda i: (i, 0)),
      ],
      out_shape=[
          jax.ShapeDtypeStruct((N, D_H), jnp.float32),
          jax.ShapeDtypeStruct((N, 1), jnp.float32),
      ],
  )(x, w_in, d0, d1)


def _agg_spec():
  return pl.BlockSpec((1, _R2, D_H), lambda j: (j // _HB, j % _HB, 0))


def _b2_body(agg_ref, g_ref, dinv_ref, b_ref, w_ref, o_ref):
  a = agg_ref[0]
  dinv = dinv_ref[...]
  h = jax.nn.relu(dinv * (a + g_ref[...]) + b_ref[...])
  g2 = jnp.dot(h, w_ref[...], preferred_element_type=jnp.float32)
  o_ref[...] = g2 * dinv


def _tc_b2(agg, g, dinv, b, w):
  return pl.pallas_call(
      _b2_body,
      grid=(_G2,),
      in_specs=[
          _agg_spec(),
          pl.BlockSpec((_R2, D_H), lambda j: (j, 0)),
          pl.BlockSpec((_R2, 1), lambda j: (j, 0)),
          pl.BlockSpec((1, D_H), lambda j: (0, 0)),
          pl.BlockSpec((D_H, D_H), lambda j: (0, 0)),
      ],
      out_specs=pl.BlockSpec((_R2, D_H), lambda j: (j, 0)),
      out_shape=jax.ShapeDtypeStruct((N, D_H), jnp.float32),
  )(agg, g, dinv, b, w)


def _b3_body(agg_ref, g_ref, dinv_ref, b_ref, o_ref):
  dinv = dinv_ref[...]
  h = jax.nn.relu(dinv * (agg_ref[0] + g_ref[...]) + b_ref[...])
  o_ref[...] = h * dinv


def _tc_b3(agg, g, dinv, b):
  return pl.pallas_call(
      _b3_body,
      grid=(_G2,),
      in_specs=[
          _agg_spec(),
          pl.BlockSpec((_R2, D_H), lambda j: (j, 0)),
          pl.BlockSpec((_R2, 1), lambda j: (j, 0)),
          pl.BlockSpec((1, D_H), lambda j: (0, 0)),
      ],
      out_specs=pl.BlockSpec((_R2, D_H), lambda j: (j, 0)),
      out_shape=jax.ShapeDtypeStruct((N, D_H), jnp.float32),
  )(agg, g, dinv, b)


def _b4_body(agg_ref, g_ref, dinv_ref, w_ref, b_ref, o_ref):
  dinv = dinv_ref[...]
  s3 = dinv * (agg_ref[0] + g_ref[...])
  o_ref[...] = jnp.dot(
      s3, w_ref[...], preferred_element_type=jnp.float32) + b_ref[...]


def _tc_b4(agg, g, dinv, w, b):
  return pl.pallas_call(
      _b4_body,
      grid=(_G2,),
      in_specs=[
          _agg_spec(),
          pl.BlockSpec((_R2, D_H), lambda j: (j, 0)),
          pl.BlockSpec((_R2, 1), lambda j: (j, 0)),
          pl.BlockSpec((D_H, D_OUT), lambda j: (0, 0)),
          pl.BlockSpec((1, D_OUT), lambda j: (0, 0)),
      ],
      out_specs=pl.BlockSpec((_R2, D_OUT), lambda j: (j, 0)),
      out_shape=jax.ShapeDtypeStruct((N, D_OUT), jnp.float32),
  )(agg, g, dinv, w, b)


def kernel(x, edge_index, W_in, b_in, W_h, b_h, W_out, b_out):
  src = edge_index[0]
  dst = edge_index[1]
  pad = EPAD - E
  src1d = jnp.concatenate([src, jnp.zeros((pad,), jnp.int32)])
  dst1d = jnp.concatenate([dst, jnp.full((pad,), N, jnp.int32)])
  dst2d = dst1d.reshape(CHUNKS, LANES)

  psrc, pdst, pcnt = _sc_partition(src1d, dst1d)
  psrc = psrc.reshape(NC, NW, SEG_CHUNKS, LANES)
  pdst = pdst.reshape(NC, NW, SEG_CHUNKS, LANES)

  d0, d1 = _sc_degree(dst2d)
  g1, dinv = _tc_b1(x, W_in, d0.reshape(DEG_NPAD, 1), d1.reshape(DEG_NPAD, 1))

  agg1 = _sc_agg(g1, psrc, pdst, pcnt)
  g2 = _tc_b2(agg1, g1, dinv, b_in.reshape(1, D_H), W_h)

  agg2 = _sc_agg(g2, psrc, pdst, pcnt)
  g3 = _tc_b3(agg2, g2, dinv, b_h.reshape(1, D_H))

  agg3 = _sc_agg(g3, psrc, pdst, pcnt)
  out = _tc_b4(agg3, g3, dinv, W_out, b_out.reshape(1, D_OUT))
  return out
